# baseline (speedup 1.0000x reference)
import functools

import jax
import jax.numpy as jnp
from jax import lax
from jax.experimental import pallas as pl
from jax.experimental.pallas import tpu as pltpu
from jax.experimental.pallas import tpu_sc as plsc

B = 512
D_MODEL = 1024
D_HIDDEN = 16384
K = 128
DEAD_THRESHOLD = 10000000.0 / 256.0

BHE = 2048
NHE = D_HIDDEN // BHE
BHD = 2048
NHD = D_HIDDEN // BHD

NW = 32
RPW = B // NW
NSLAB = RPW // 2


def _encode_body(x_ref, wenc_ref, benc_ref, bpre_ref,
                 keys_ref, mu_ref, std_ref, xs_ref):
    s = pl.program_id(0)

    @pl.when(s == 0)
    def _():
        x = x_ref[...]
        mu = jnp.mean(x, axis=-1, keepdims=True)
        xc = x - mu
        var = jnp.sum(xc * xc, axis=-1, keepdims=True) / (D_MODEL - 1)
        std = jnp.sqrt(var)
        mu_ref[...] = mu
        std_ref[...] = std
        xs_ref[...] = xc / (std + 1e-5) - bpre_ref[...]

    pre = (
        jnp.dot(xs_ref[...], wenc_ref[...], preferred_element_type=jnp.float32)
        + benc_ref[...]
    )
    keys_ref[...] = lax.bitcast_convert_type(pre, jnp.int32)


_PASS_SHIFTS = (23, 15, 7, 0)
_PASS_MASKS = (255, 255, 255, 127)


def _sc_topk_body(keys_hbm, thr_hbm, slab_a, slab_b, hist, thrbuf,
                  sem_a, sem_b):
    wid = lax.axis_index("s") * 2 + lax.axis_index("c")
    base = wid * RPW
    iota16 = lax.iota(jnp.int32, 16)
    ones16 = jnp.ones((16,), jnp.float32)

    def zero_hist():
        def z(v, carry):
            hist[pl.ds(v * 16, 16)] = jnp.zeros((16,), jnp.float32)
            return carry
        lax.fori_loop(0, 32, z, jnp.int32(0))

    def scan_row(r, kneed):
        roff = r * 256

        def sweep(v, carry):
            vi = 15 - v
            h = hist[pl.ds(roff + vi * 16, 16)]
            total = jnp.sum(h)
            s = lax.rev(jnp.cumsum(lax.rev(h, (0,))), (0,))
            hist[pl.ds(roff + vi * 16, 16)] = s + carry
            return carry + total
        lax.fori_loop(0, 16, sweep, jnp.float32(0))

        def findb(v, b):
            srow = hist[pl.ds(roff + v * 16, 16)]
            cand = jnp.max(jnp.where(srow >= kneed, iota16 + v * 16, -1))
            return jnp.maximum(b, cand)
        b = lax.fori_loop(0, 16, findb, jnp.int32(-1))

        nxt = b + 1
        start = jnp.minimum((nxt // 16) * 16, 240)
        w = hist[pl.ds(roff + start, 16)]
        above = jnp.sum(jnp.where(iota16 == (nxt - start), w, 0.0))
        above = jnp.where((b < 0) | (nxt >= 256), 0.0, above)
        return b, above

    def select_two(slab):
        kneed = [jnp.float32(K) for _ in range(2)]
        bsel = [[], []]
        for p, (shift, mmask) in enumerate(zip(_PASS_SHIFTS, _PASS_MASKS)):
            zero_hist()
            for r in (0, 1):
                prev = list(bsel[r])

                def hbody(j, carry, _r=r, _prev=prev, _shift=shift,
                          _mmask=mmask):
                    for u in range(4):
                        v = slab[_r, pl.ds((j * 4 + u) * 16, 16)]
                        m = v >= 0
                        for q, bq in enumerate(_prev):
                            fld = lax.shift_right_arithmetic(
                                v, _PASS_SHIFTS[q])
                            m = m & ((fld & _PASS_MASKS[q]) == bq)
                        idx = lax.shift_right_arithmetic(v, _shift) & _mmask
                        idx = jnp.where(m, idx + _r * 256, _r * 256)
                        plsc.addupdate_scatter(hist, [idx], ones16, mask=m)
                    return carry
                lax.fori_loop(0, D_HIDDEN // 16 // 4, hbody, jnp.int32(0))
            for r in (0, 1):
                b, above = scan_row(r, kneed[r])
                kneed[r] = kneed[r] - above
                bsel[r].append(b)

        thrs = []
        for r in (0, 1):
            b0, b1, b2, b3 = bsel[r]
            thr = (b0 << 23) | (b1 << 15) | (b2 << 7) | b3
            thrs.append(jnp.where(b0 < 0, 0, thr))
        return thrs

    pltpu.async_copy(keys_hbm.at[pl.ds(base, 2)], slab_a, sem_a)

    def pair_body(i, thrvec):
        pltpu.make_async_copy(
            keys_hbm.at[pl.ds(base, 2)], slab_a, sem_a
        ).wait()
        pltpu.async_copy(
            keys_hbm.at[pl.ds(base + (2 * i + 1) * 2, 2)], slab_b, sem_b
        )
        thr0, thr1 = select_two(slab_a)
        row = 4 * i
        thrvec = jnp.where(iota16 == row, thr0, thrvec)
        thrvec = jnp.where(iota16 == row + 1, thr1, thrvec)

        pltpu.make_async_copy(
            keys_hbm.at[pl.ds(base, 2)], slab_b, sem_b
        ).wait()
        nxt_slab = jnp.minimum(2 * i + 2, NSLAB - 1)
        pltpu.async_copy(
            keys_hbm.at[pl.ds(base + nxt_slab * 2, 2)], slab_a, sem_a
        )
        thr2, thr3 = select_two(slab_b)
        thrvec = jnp.where(iota16 == row + 2, thr2, thrvec)
        thrvec = jnp.where(iota16 == row + 3, thr3, thrvec)
        return thrvec

    thrvec = lax.fori_loop(
        0, NSLAB // 2, pair_body, jnp.zeros((16,), jnp.int32)
    )
    pltpu.make_async_copy(keys_hbm.at[pl.ds(base, 2)], slab_a, sem_a).wait()
    thrbuf[...] = thrvec
    pltpu.sync_copy(thrbuf, thr_hbm.at[pl.ds(base, 16)])


def _decode_body(keys_ref, thr_ref, wdec_ref, mu_ref, std_ref, bpre_ref,
                 stats_ref, out_ref, ndead_ref, featzero_ref):
    h = pl.program_id(0)
    key = keys_ref[...]
    thr = thr_ref[...]
    lat = jnp.where(
        key >= thr, lax.bitcast_convert_type(key, jnp.float32), 0.0
    )
    part = jnp.dot(lat, wdec_ref[...], preferred_element_type=jnp.float32)

    chunk_any = jnp.max(
        (key >= jnp.maximum(thr, 1)).astype(jnp.int32), axis=0, keepdims=True
    )
    featzero_ref[:, pl.ds(h * BHD, BHD)] = 1 - chunk_any

    @pl.when(h == 0)
    def _():
        out_ref[...] = part

    @pl.when(h > 0)
    def _():
        out_ref[...] = out_ref[...] + part

    @pl.when(h == NHD - 1)
    def _():
        out_ref[...] = (
            (out_ref[...] + bpre_ref[...]) * std_ref[...] + mu_ref[...]
        )
        stats_new = stats_ref[...] * featzero_ref[...] + 1
        dead = (stats_new.astype(jnp.float32) > DEAD_THRESHOLD)
        ndead_ref[0, 0] = jnp.sum(dead.astype(jnp.int32))


_sc_topk = functools.partial(
    pl.kernel,
    out_type=jax.ShapeDtypeStruct((B,), jnp.int32),
    mesh=plsc.VectorSubcoreMesh(core_axis_name="c", subcore_axis_name="s"),
    scratch_types=[
        pltpu.VMEM((2, D_HIDDEN), jnp.int32),
        pltpu.VMEM((2, D_HIDDEN), jnp.int32),
        pltpu.VMEM((512,), jnp.float32),
        pltpu.VMEM((16,), jnp.int32),
        pltpu.SemaphoreType.DMA,
        pltpu.SemaphoreType.DMA,
    ],
    compiler_params=pltpu.CompilerParams(needs_layout_passes=False),
)(_sc_topk_body)


@jax.jit
def kernel(x, w_enc, w_dec, b_enc, b_pre, stats_last_nonzero):
    b_enc2 = b_enc.reshape(1, D_HIDDEN)
    b_pre2 = b_pre.reshape(1, D_MODEL)
    stats2 = stats_last_nonzero.reshape(1, D_HIDDEN)

    keys, mu, std = pl.pallas_call(
        _encode_body,
        grid=(NHE,),
        in_specs=[
            pl.BlockSpec((B, D_MODEL), lambda s: (0, 0)),
            pl.BlockSpec((D_MODEL, BHE), lambda s: (0, s)),
            pl.BlockSpec((1, BHE), lambda s: (0, s)),
            pl.BlockSpec((1, D_MODEL), lambda s: (0, 0)),
        ],
        out_specs=[
            pl.BlockSpec((B, BHE), lambda s: (0, s)),
            pl.BlockSpec((B, 1), lambda s: (0, 0)),
            pl.BlockSpec((B, 1), lambda s: (0, 0)),
        ],
        out_shape=[
            jax.ShapeDtypeStruct((B, D_HIDDEN), jnp.int32),
            jax.ShapeDtypeStruct((B, 1), jnp.float32),
            jax.ShapeDtypeStruct((B, 1), jnp.float32),
        ],
        scratch_shapes=[pltpu.VMEM((B, D_MODEL), jnp.float32)],
        compiler_params=pltpu.CompilerParams(
            dimension_semantics=("arbitrary",),
        ),
    )(x, w_enc, b_enc2, b_pre2)

    thr = _sc_topk(keys)

    recons, ndead = pl.pallas_call(
        _decode_body,
        grid=(NHD,),
        in_specs=[
            pl.BlockSpec((B, BHD), lambda h: (0, h)),
            pl.BlockSpec((B, 1), lambda h: (0, 0)),
            pl.BlockSpec((BHD, D_MODEL), lambda h: (h, 0)),
            pl.BlockSpec((B, 1), lambda h: (0, 0)),
            pl.BlockSpec((B, 1), lambda h: (0, 0)),
            pl.BlockSpec((1, D_MODEL), lambda h: (0, 0)),
            pl.BlockSpec((1, D_HIDDEN), lambda h: (0, 0)),
        ],
        out_specs=[
            pl.BlockSpec((B, D_MODEL), lambda h: (0, 0)),
            pl.BlockSpec(memory_space=pltpu.SMEM),
        ],
        out_shape=[
            jax.ShapeDtypeStruct((B, D_MODEL), jnp.float32),
            jax.ShapeDtypeStruct((1, 1), jnp.int32),
        ],
        scratch_shapes=[pltpu.VMEM((1, D_HIDDEN), jnp.int32)],
        compiler_params=pltpu.CompilerParams(
            dimension_semantics=("arbitrary",),
        ),
    )(keys, thr.reshape(B, 1), w_dec, mu, std, b_pre2, stats2)

    return (recons, ndead[0, 0])


# speedup vs baseline: 3.9401x; 3.9401x over previous
import functools

import jax
import jax.numpy as jnp
from jax.experimental import pallas as pl
from jax.experimental.pallas import tpu as pltpu

B = 512
D_MODEL = 1024
D_HIDDEN = 16384
K = 128
DEAD_THRESHOLD = 10000000.0 / 256.0

BH = 1024
NH = D_HIDDEN // BH
BC = 128
NC = B // BC


def _fused_body(x_ref, wenc_ref, wdec_ref, benc_ref, bpre_ref, stats_ref,
                out_ref, ndead_ref,
                keys_ref, xs_ref, mu_ref, std_ref, thr_ref,
                featzero_ref):
    s = pl.program_id(0)

    @pl.when(s == 0)
    def _():
        x = x_ref[...]
        mu = jnp.mean(x, axis=-1, keepdims=True)
        xc = x - mu
        var = jnp.sum(xc * xc, axis=-1, keepdims=True) / (D_MODEL - 1)
        std = jnp.sqrt(var)
        mu_ref[...] = mu
        std_ref[...] = std
        xs_ref[...] = xc / (std + 1e-5) - bpre_ref[...]

    @pl.when(s < NH)
    def _():
        pre = (
            jnp.dot(xs_ref[...], wenc_ref[...], preferred_element_type=jnp.float32)
            + benc_ref[...]
        )
        keys_ref[:, pl.ds(s * BH, BH)] = jax.lax.bitcast_convert_type(
            pre, jnp.int32
        )

    @pl.when((s >= NH) & (s < NH + NC))
    def _():
        c = s - NH
        rows = pl.ds(c * BC, BC)

        thr0 = jnp.zeros((BC, 1), jnp.int32)

        def bit_step(i, thr):
            bit = jnp.int32(1) << (jnp.int32(30) - i)
            cand = thr | bit
            cnt = jnp.sum(
                (keys_ref[rows, :] >= cand).astype(jnp.int32),
                axis=1, keepdims=True,
            )
            return jnp.where(cnt >= K, cand, thr)

        thr_eff = jax.lax.fori_loop(0, 31, bit_step, thr0)
        thr_ref[rows, :] = thr_eff

        chunk_any = jnp.max(
            (keys_ref[rows, :] >= jnp.maximum(thr_eff, 1)).astype(jnp.int32),
            axis=0, keepdims=True,
        )

        @pl.when(c == 0)
        def _():
            featzero_ref[...] = 1 - chunk_any

        @pl.when(c > 0)
        def _():
            featzero_ref[...] = featzero_ref[...] * (1 - chunk_any)

        @pl.when(c == NC - 1)
        def _():
            stats_new = stats_ref[...] * featzero_ref[...] + 1
            dead = (stats_new.astype(jnp.float32) > DEAD_THRESHOLD)
            ndead_ref[0, 0] = jnp.sum(dead.astype(jnp.int32))

    @pl.when(s >= NH + NC)
    def _():
        h = s - (NH + NC)
        key = keys_ref[:, pl.ds(h * BH, BH)]
        lat = jnp.where(
            key >= thr_ref[...],
            jax.lax.bitcast_convert_type(key, jnp.float32),
            0.0,
        )
        part = jnp.dot(lat, wdec_ref[...], preferred_element_type=jnp.float32)

        @pl.when(h == 0)
        def _():
            out_ref[...] = part

        @pl.when(h > 0)
        def _():
            out_ref[...] = out_ref[...] + part

        @pl.when(h == NH - 1)
        def _():
            out_ref[...] = (
                (out_ref[...] + bpre_ref[...]) * std_ref[...] + mu_ref[...]
            )


@jax.jit
def kernel(x, w_enc, w_dec, b_enc, b_pre, stats_last_nonzero):
    b_enc2 = b_enc.reshape(1, D_HIDDEN)
    b_pre2 = b_pre.reshape(1, D_MODEL)
    stats2 = stats_last_nonzero.reshape(1, D_HIDDEN)

    recons, ndead = pl.pallas_call(
        _fused_body,
        grid=(NH + NC + NH,),
        in_specs=[
            pl.BlockSpec((B, D_MODEL), lambda s: (0, 0)),
            pl.BlockSpec((D_MODEL, BH),
                         lambda s: (0, jnp.where(s < NH, s, NH - 1))),
            pl.BlockSpec((BH, D_MODEL),
                         lambda s: (jnp.where(s >= NH + NC, s - (NH + NC), 0), 0)),
            pl.BlockSpec((1, BH),
                         lambda s: (0, jnp.where(s < NH, s, NH - 1))),
            pl.BlockSpec((1, D_MODEL), lambda s: (0, 0)),
            pl.BlockSpec((1, D_HIDDEN), lambda s: (0, 0)),
        ],
        out_specs=[
            pl.BlockSpec((B, D_MODEL), lambda s: (0, 0)),
            pl.BlockSpec(memory_space=pltpu.SMEM),
        ],
        out_shape=[
            jax.ShapeDtypeStruct((B, D_MODEL), jnp.float32),
            jax.ShapeDtypeStruct((1, 1), jnp.int32),
        ],
        scratch_shapes=[
            pltpu.VMEM((B, D_HIDDEN), jnp.int32),
            pltpu.VMEM((B, D_MODEL), jnp.float32),
            pltpu.VMEM((B, 1), jnp.float32),
            pltpu.VMEM((B, 1), jnp.float32),
            pltpu.VMEM((B, 1), jnp.int32),
            pltpu.VMEM((1, D_HIDDEN), jnp.int32),
        ],
        compiler_params=pltpu.CompilerParams(
            dimension_semantics=("arbitrary",),
            vmem_limit_bytes=63 * 1024 * 1024,
        ),
    )(x, w_enc, w_dec, b_enc2, b_pre2, stats2)

    return (recons, ndead[0, 0])
